# baseline (speedup 1.0000x reference)
import functools

import jax
import jax.numpy as jnp
from jax import lax
from jax.experimental import pallas as pl
from jax.experimental.pallas import tpu as pltpu
from jax.experimental.pallas import tpu_sc as plsc

VOCAB = 151936
HIDDEN = 1536
BATCH = 4
SEQ = 2048

_info = plsc.get_sparse_core_info()
_NC, _NS = _info.num_cores, _info.num_subcores
_NW = _NC * _NS
_NTOT = BATCH * SEQ
_BPW = _NTOT // _NW
_CHUNK = 32
_NCHUNK = _BPW // _CHUNK


@functools.partial(
    pl.kernel,
    mesh=plsc.VectorSubcoreMesh(core_axis_name="c", subcore_axis_name="s"),
    out_type=jax.ShapeDtypeStruct((_NTOT, HIDDEN), jnp.float32),
    scratch_types=[
        pltpu.VMEM((_BPW,), jnp.int32),
        pltpu.VMEM((_CHUNK, HIDDEN), jnp.float32),
        pltpu.VMEM((_CHUNK, HIDDEN), jnp.float32),
        pltpu.SemaphoreType.DMA,
        pltpu.SemaphoreType.DMA,
    ],
)
def _gather_rows(table_hbm, ids_hbm, out_hbm, idx_v, rows_a, rows_b, sem_a, sem_b):
    wid = lax.axis_index("s") * _NC + lax.axis_index("c")
    base = wid * _BPW
    pltpu.sync_copy(ids_hbm.at[pl.ds(base, _BPW)], idx_v)
    bufs = (rows_a, rows_b)
    sems = (sem_a, sem_b)
    copies = [None] * _NCHUNK
    copies[0] = pltpu.async_copy(
        table_hbm.at[idx_v.at[pl.ds(0, _CHUNK)]], bufs[0], sems[0])
    for j in range(_NCHUNK):
        if j + 1 < _NCHUNK:
            copies[j + 1] = pltpu.async_copy(
                table_hbm.at[idx_v.at[pl.ds((j + 1) * _CHUNK, _CHUNK)]],
                bufs[(j + 1) % 2], sems[(j + 1) % 2])
        copies[j].wait()
        pltpu.sync_copy(bufs[j % 2], out_hbm.at[pl.ds(base + j * _CHUNK, _CHUNK)])


def kernel(input_ids, attention_mask, table):
    ids_flat = input_ids.reshape(_NTOT)
    out_flat = _gather_rows(table, ids_flat)
    hidden_states = out_flat.reshape(BATCH, SEQ, HIDDEN)
    seq_lengths = attention_mask.sum(axis=1) - 1
    last_hidden = hidden_states[jnp.arange(BATCH), seq_lengths]
    return (hidden_states, last_hidden)


# speedup vs baseline: 1.0120x; 1.0120x over previous
import functools

import jax
import jax.numpy as jnp
from jax import lax
from jax.experimental import pallas as pl
from jax.experimental.pallas import tpu as pltpu
from jax.experimental.pallas import tpu_sc as plsc

VOCAB = 151936
HIDDEN = 1536
BATCH = 4
SEQ = 2048

_info = plsc.get_sparse_core_info()
_NC, _NS = _info.num_cores, _info.num_subcores
_NW = _NC * _NS
_NTOT = BATCH * SEQ
_BPW = _NTOT // _NW
_CHUNK = 32
_NCHUNK = _BPW // _CHUNK


@functools.partial(
    pl.kernel,
    mesh=plsc.VectorSubcoreMesh(core_axis_name="c", subcore_axis_name="s"),
    out_type=(
        jax.ShapeDtypeStruct((_NTOT, HIDDEN), jnp.float32),
        jax.ShapeDtypeStruct((BATCH, HIDDEN), jnp.float32),
    ),
    scratch_types=[
        pltpu.VMEM((_BPW,), jnp.int32),
        pltpu.VMEM((8,), jnp.int32),
        pltpu.VMEM((_CHUNK, HIDDEN), jnp.float32),
        pltpu.VMEM((_CHUNK, HIDDEN), jnp.float32),
        pltpu.VMEM((8, HIDDEN), jnp.float32),
        pltpu.SemaphoreType.DMA,
        pltpu.SemaphoreType.DMA,
        pltpu.SemaphoreType.DMA,
    ],
)
def _gather_rows(table_hbm, ids_hbm, last_ids_hbm, out_hbm, last_hbm,
                 idx_v, lidx_v, rows_a, rows_b, last_rows, sem_a, sem_b, sem_l):
    wid = lax.axis_index("s") * _NC + lax.axis_index("c")
    base = wid * _BPW
    pltpu.sync_copy(ids_hbm.at[pl.ds(base, _BPW)], idx_v)
    bufs = (rows_a, rows_b)
    sems = (sem_a, sem_b)
    copies = [None] * _NCHUNK
    copies[0] = pltpu.async_copy(
        table_hbm.at[idx_v.at[pl.ds(0, _CHUNK)]], bufs[0], sems[0])
    @pl.when(wid == _NW - 1)
    def _():
        pltpu.sync_copy(last_ids_hbm, lidx_v)
        pltpu.async_copy(table_hbm.at[lidx_v], last_rows, sem_l).wait()
        pltpu.sync_copy(last_rows.at[pl.ds(0, BATCH)], last_hbm)

    for j in range(_NCHUNK):
        if j + 1 < _NCHUNK:
            copies[j + 1] = pltpu.async_copy(
                table_hbm.at[idx_v.at[pl.ds((j + 1) * _CHUNK, _CHUNK)]],
                bufs[(j + 1) % 2], sems[(j + 1) % 2])
        copies[j].wait()
        pltpu.sync_copy(bufs[j % 2], out_hbm.at[pl.ds(base + j * _CHUNK, _CHUNK)])


def kernel(input_ids, attention_mask, table):
    ids_flat = input_ids.reshape(_NTOT)
    seq_lengths = attention_mask.sum(axis=1) - 1
    last_ids = jnp.take_along_axis(input_ids, seq_lengths[:, None], axis=1)
    last_ids8 = jnp.concatenate([last_ids[:, 0], jnp.zeros((4,), jnp.int32)])
    out_flat, last_hidden = _gather_rows(table, ids_flat, last_ids8)
    hidden_states = out_flat.reshape(BATCH, SEQ, HIDDEN)
    return (hidden_states, last_hidden)
